# Initial kernel scaffold; baseline (speedup 1.0000x reference)
import jax
import jax.numpy as jnp
from jax.experimental import pallas as pl


def kernel(x, edge_index, edge_weight, W1, b1, W2, b2):
    raise NotImplementedError("write your pallas kernel here")


# speedup vs baseline: 4.3754x; 4.3754x over previous
import functools

import jax
import jax.numpy as jnp
from jax import lax
from jax.experimental import pallas as pl
from jax.experimental.pallas import tpu as pltpu
from jax.experimental.pallas import tpu_sc as plsc

N = 10000
E = 320000
F = 128

NC = 2
NS = 16
NW = NC * NS
L = 16

EPW = E // NW
C = 80
NCHUNK = EPW // C
N_PAD = 10240
RPT = N_PAD // NS


def _spmm_entry(sup_hbm, src_hbm, dst_hbm, w_hbm, zero_hbm, parts_hbm,
                src_v, w_v, dst_c, rows_v, acc, gsem):
    c = lax.axis_index("c")
    s = lax.axis_index("s")
    wid = c * NS + s
    base = wid * EPW

    if True:
        pltpu.sync_copy(zero_hbm.at[pl.ds(s * RPT, RPT)],
                        acc.at[pl.ds(s * RPT, RPT)])
        pltpu.sync_copy(src_hbm.at[pl.ds(base, EPW)], src_v)
        pltpu.sync_copy(w_hbm.at[pl.ds(base, EPW)], w_v)
        plsc.subcore_barrier()

        def chunk_body(i, carry):
            off = i * C
            pltpu.sync_copy(dst_hbm.at[pl.ds(base + off, C)], dst_c)
            pltpu.async_copy(sup_hbm.at[src_v.at[pl.ds(off, C)]], rows_v,
                             gsem).wait()

            def row_body(r, rcarry):
                wb = plsc.load_gather(w_v, [lax.broadcast(off + r, (L,))])
                for j in range(F // L):
                    sl = (r, pl.ds(j * L, L))
                    rows_v[sl] = rows_v[sl] * wb
                return rcarry

            lax.fori_loop(0, C, row_body, 0)
            pltpu.sync_copy(rows_v, acc.at[dst_c], add=True)
            return carry

        lax.fori_loop(0, NCHUNK, chunk_body, 0)
        plsc.subcore_barrier()
        pltpu.sync_copy(acc.at[pl.ds(s * RPT, RPT)],
                        parts_hbm.at[c, pl.ds(s * RPT, RPT)])


_spmm = pl.kernel(
    _spmm_entry,
    out_type=jax.ShapeDtypeStruct((NC, N_PAD, F), jnp.float32),
    mesh=plsc.VectorSubcoreMesh(core_axis_name="c", subcore_axis_name="s"),
    compiler_params=pltpu.CompilerParams(needs_layout_passes=False),
    scratch_types=[
        pltpu.VMEM((EPW,), jnp.int32),
        pltpu.VMEM((EPW,), jnp.float32),
        pltpu.VMEM((C,), jnp.int32),
        pltpu.VMEM((C, F), jnp.float32),
        pltpu.VMEM_SHARED((N_PAD, F), jnp.float32),
        pltpu.SemaphoreType.DMA,
    ],
)


def _mm_kernel(x_ref, w_ref, o_ref):
    o_ref[...] = jnp.dot(x_ref[...], w_ref[...],
                         preferred_element_type=jnp.float32)


def _mid_kernel(p_ref, b_ref, w_ref, o_ref):
    h = jnp.maximum(p_ref[0] + p_ref[1] + b_ref[...], 0.0)
    o_ref[...] = jnp.dot(h, w_ref[...], preferred_element_type=jnp.float32)


def _out_kernel(p_ref, b_ref, o_ref):
    o_ref[...] = jnp.maximum(p_ref[0] + p_ref[1] + b_ref[...], 0.0)


_BM = 2000


def _mm(x, w):
    return pl.pallas_call(
        _mm_kernel,
        grid=(N // _BM,),
        in_specs=[pl.BlockSpec((_BM, F), lambda i: (i, 0)),
                  pl.BlockSpec((F, F), lambda i: (0, 0))],
        out_specs=pl.BlockSpec((_BM, F), lambda i: (i, 0)),
        out_shape=jax.ShapeDtypeStruct((N, F), jnp.float32),
    )(x, w)


def _mid(parts, b, w):
    return pl.pallas_call(
        _mid_kernel,
        grid=(N // _BM,),
        in_specs=[pl.BlockSpec((NC, _BM, F), lambda i: (0, i, 0)),
                  pl.BlockSpec((1, F), lambda i: (0, 0)),
                  pl.BlockSpec((F, F), lambda i: (0, 0))],
        out_specs=pl.BlockSpec((_BM, F), lambda i: (i, 0)),
        out_shape=jax.ShapeDtypeStruct((N, F), jnp.float32),
    )(parts, b.reshape(1, F), w)


def _final(parts, b):
    return pl.pallas_call(
        _out_kernel,
        grid=(N // _BM,),
        in_specs=[pl.BlockSpec((NC, _BM, F), lambda i: (0, i, 0)),
                  pl.BlockSpec((1, F), lambda i: (0, 0))],
        out_specs=pl.BlockSpec((_BM, F), lambda i: (i, 0)),
        out_shape=jax.ShapeDtypeStruct((N, F), jnp.float32),
    )(parts, b.reshape(1, F))


def kernel(x, edge_index, edge_weight, W1, b1, W2, b2):
    src = edge_index[0]
    dst = edge_index[1]
    zeros = jnp.zeros((N_PAD, F), jnp.float32)
    s1 = _mm(x, W1)
    parts1 = _spmm(s1, src, dst, edge_weight, zeros)
    s2 = _mid(parts1, b1, W2)
    parts2 = _spmm(s2, src, dst, edge_weight, zeros)
    return _final(parts2, b2)
